# baseline (speedup 1.0000x reference)
import functools

import jax
import jax.numpy as jnp
from jax import lax
from jax.experimental import pallas as pl
from jax.experimental.pallas import tpu as pltpu
from jax.experimental.pallas import tpu_sc as plsc

H = 4096
W = 4096
N = 4194304

_info = plsc.get_sparse_core_info()
NC, NS, L = _info.num_cores, _info.num_subcores, _info.num_lanes
NW = NC * NS
PER_TILE = N // NW
C = 4096
CHUNKS = PER_TILE // C
MAXX = H - 1
MAXY = W - 1


def _make_sc_call():
    mesh = plsc.VectorSubcoreMesh(core_axis_name="c", subcore_axis_name="s")

    @functools.partial(
        pl.kernel,
        mesh=mesh,
        out_type=(
            jax.ShapeDtypeStruct((N,), jnp.float32),
            jax.ShapeDtypeStruct((NW, L), jnp.float32),
        ),
        scratch_types=[
            pltpu.VMEM((C,), jnp.float32),
            pltpu.VMEM((C,), jnp.float32),
            pltpu.VMEM((C,), jnp.int32),
            pltpu.VMEM((C,), jnp.int32),
            pltpu.VMEM((C,), jnp.int32),
            pltpu.VMEM((C,), jnp.int32),
            pltpu.VMEM((C,), jnp.float32),
            pltpu.VMEM((C,), jnp.float32),
            pltpu.VMEM((C,), jnp.float32),
            pltpu.VMEM((C,), jnp.float32),
            pltpu.VMEM((C,), jnp.float32),
            pltpu.VMEM((16,), jnp.float32),
            pltpu.SemaphoreType.DMA,
        ],
    )
    def sc_kernel(x_hbm, y_hbm, sdf_hbm, out_hbm, part_hbm,
                  xv, yv, i11, i21, i12, i22, v11, v21, v12, v22, ov,
                  accv, gsem):
        wid = lax.axis_index("s") * NC + lax.axis_index("c")
        base0 = wid * PER_TILE

        def chunk_body(c, carry):
            base = base0 + c * C
            pltpu.sync_copy(x_hbm.at[pl.ds(base, C)], xv)
            pltpu.sync_copy(y_hbm.at[pl.ds(base, C)], yv)

            def idx_body(i, carry2):
                o = i * L
                xs = xv[pl.ds(o, L)]
                ys = yv[pl.ds(o, L)]
                x1 = jnp.minimum(xs.astype(jnp.int32), MAXX)
                y1 = jnp.minimum(ys.astype(jnp.int32), MAXY)
                x2 = jnp.minimum(x1 + 1, MAXX)
                y2 = jnp.minimum(y1 + 1, MAXY)
                s1 = x1 << 12
                s2 = x2 << 12
                i11[pl.ds(o, L)] = s1 + y1
                i21[pl.ds(o, L)] = s2 + y1
                i12[pl.ds(o, L)] = s1 + y2
                i22[pl.ds(o, L)] = s2 + y2
                return carry2

            lax.fori_loop(0, C // L, idx_body, 0)

            d1 = pltpu.async_copy(sdf_hbm.at[i11], v11, gsem)
            d2 = pltpu.async_copy(sdf_hbm.at[i21], v21, gsem)
            d3 = pltpu.async_copy(sdf_hbm.at[i12], v12, gsem)
            d4 = pltpu.async_copy(sdf_hbm.at[i22], v22, gsem)
            d1.wait()
            d2.wait()
            d3.wait()
            d4.wait()

            def blend_body(i, carry2):
                o = i * L
                xs = xv[pl.ds(o, L)]
                ys = yv[pl.ds(o, L)]
                x1 = jnp.minimum(xs.astype(jnp.int32), MAXX)
                y1 = jnp.minimum(ys.astype(jnp.int32), MAXY)
                t = xs - x1.astype(jnp.float32)
                u = ys - y1.astype(jnp.float32)
                a = v11[pl.ds(o, L)]
                b = v21[pl.ds(o, L)]
                cc = v12[pl.ds(o, L)]
                dd = v22[pl.ds(o, L)]
                top = a + t * (b - a)
                bot = cc + t * (dd - cc)
                s = top + u * (bot - top)
                s = jnp.minimum(jnp.maximum(s, -1000.0), 0.0)
                ov[pl.ds(o, L)] = s
                return carry2 + s

            acc = lax.fori_loop(0, C // L, blend_body, carry)
            pltpu.sync_copy(ov, out_hbm.at[pl.ds(base, C)])
            return acc

        acc = lax.fori_loop(0, CHUNKS, chunk_body, jnp.zeros((L,), jnp.float32))
        accv[...] = acc
        pltpu.sync_copy(accv, part_hbm.at[wid])

    return sc_kernel


_sc_call = _make_sc_call()


def kernel(x, y, sdf_array):
    sdf_flat = sdf_array.reshape(-1)
    out_vals, partials = _sc_call(x, y, sdf_flat)
    loss = jnp.sum(partials)
    return (loss, out_vals)
